# baseline (speedup 1.0000x reference)
import functools

import jax
import jax.numpy as jnp
from jax import lax
from jax.experimental import pallas as pl
from jax.experimental.pallas import tpu as pltpu
from jax.experimental.pallas import tpu_sc as plsc


def _sc_broadcast_add(x_table, y_table, B, X, Y, D):
    info = plsc.get_sparse_core_info()
    NC, NS, L = info.num_cores, info.num_subcores, info.num_lanes
    mesh = plsc.VectorSubcoreMesh(core_axis_name="c", subcore_axis_name="s")

    @functools.partial(
        pl.kernel,
        mesh=mesh,
        out_type=jax.ShapeDtypeStruct((B, X, Y, D), jnp.float32),
        scratch_types=[
            pltpu.VMEM((Y, D), jnp.float32),
            pltpu.VMEM((1, D), jnp.float32),
            pltpu.VMEM((Y, D), jnp.float32),
            pltpu.VMEM((Y, D), jnp.float32),
            pltpu.SemaphoreType.DMA,
        ],
    )
    def k(x_hbm, y_hbm, out_hbm, yblk, xrow, blk0, blk1, sem):
        c = lax.axis_index("c")
        s = lax.axis_index("s")
        w = s * NC + c

        @pl.when(s < NS // 2)
        def _():
            pltpu.sync_copy(y_hbm.at[pl.ds(0, Y)], yblk)
            xs = []
            for xi, blk in ((0, blk0), (1, blk1)):
                x = w * 2 + xi
                xs.append(x)
                pltpu.sync_copy(x_hbm.at[pl.ds(x, 1)], xrow)

                def row(y, carry, blk=blk):
                    for d0 in range(D // L):
                        sl = pl.ds(d0 * L, L)
                        blk[y, sl] = yblk[y, sl] + xrow[0, sl]
                    return carry

                lax.fori_loop(0, Y, row, 0)

            copies = []
            for b in range(B):
                copies.append(pltpu.async_copy(blk0, out_hbm.at[b, xs[0]], sem))
                copies.append(pltpu.async_copy(blk1, out_hbm.at[b, xs[1]], sem))
            for cp in copies:
                cp.wait()

    return k(x_table, y_table)


def kernel(inputs, x_table, y_table):
    B, X, Y, D = inputs.shape
    return _sc_broadcast_add(x_table, y_table, B, X, Y, D)


# speedup vs baseline: 1.3491x; 1.3491x over previous
import functools

import jax
import jax.numpy as jnp
from jax import lax
from jax.experimental import pallas as pl
from jax.experimental.pallas import tpu as pltpu
from jax.experimental.pallas import tpu_sc as plsc
from jax._src.pallas import mpmd as plmpmd

_B_TEC = 36
_SCS_WINDOW = 8


def _sc_broadcast_add(x_table, y_table, B, X, Y, D):
    info = plsc.get_sparse_core_info()
    NC, NS, L = info.num_cores, info.num_subcores, info.num_lanes
    vmesh = plsc.VectorSubcoreMesh(core_axis_name="c", subcore_axis_name="s")
    smesh = plsc.ScalarSubcoreMesh(axis_name="c")
    b_tec = min(_B_TEC, B)
    xh = X // NC

    @functools.partial(
        pl.kernel,
        mesh=vmesh,
        out_type=jax.ShapeDtypeStruct((X, Y, D), jnp.float32),
        scratch_types=[
            pltpu.VMEM((Y, D), jnp.float32),
            pltpu.VMEM((1, D), jnp.float32),
            pltpu.VMEM((Y, D), jnp.float32),
        ],
    )
    def make_blocks(x_hbm, y_hbm, blocks_hbm, yblk, xrow, blk):
        w = lax.axis_index("s") * NC + lax.axis_index("c")
        pltpu.sync_copy(y_hbm.at[pl.ds(0, Y)], yblk)
        pltpu.sync_copy(x_hbm.at[pl.ds(w, 1)], xrow)

        def row(y, carry):
            for d0 in range(D // L):
                sl = pl.ds(d0 * L, L)
                blk[y, sl] = yblk[y, sl] + xrow[0, sl]
            return carry

        lax.fori_loop(0, Y, row, 0)
        pltpu.sync_copy(blk, blocks_hbm.at[w])

    blocks = make_blocks(x_table, y_table)

    def tec_fn(blocks_hbm, out_hbm, sp_blocks, scs_sem, tec_blk, tec_sem):
        del sp_blocks, scs_sem
        w = lax.axis_index("s") * NC + lax.axis_index("c")
        pltpu.sync_copy(blocks_hbm.at[w], tec_blk)
        copies = [
            pltpu.async_copy(tec_blk, out_hbm.at[b, w], tec_sem)
            for b in range(b_tec)
        ]
        for cp in copies:
            cp.wait()

    def scs_fn(blocks_hbm, out_hbm, sp_blocks, scs_sem, tec_blk, tec_sem):
        del tec_blk, tec_sem
        c = lax.axis_index("c")
        for i in range(xh):
            pltpu.sync_copy(blocks_hbm.at[i * NC + c], sp_blocks.at[i])
        handles = []
        for i in range(xh):
            x = i * NC + c
            for b in range(b_tec, B):
                if len(handles) >= _SCS_WINDOW:
                    handles[len(handles) - _SCS_WINDOW].wait()
                handles.append(
                    pltpu.async_copy(sp_blocks.at[i], out_hbm.at[b, x], scs_sem)
                )
        for h in handles[-_SCS_WINDOW:]:
            h.wait()

    run = plmpmd.mpmd_map(
        [(smesh, scs_fn), (vmesh, tec_fn)],
        out_types=[jax.ShapeDtypeStruct((B, X, Y, D), jnp.float32)],
        scratch_types=[
            pltpu.VMEM_SHARED((xh, Y, D), jnp.float32),
            pltpu.SemaphoreType.DMA @ smesh,
            (pltpu.VMEM @ vmesh)((Y, D), jnp.float32),
            pltpu.SemaphoreType.DMA @ vmesh,
        ],
    )
    return run(blocks)[0]


def kernel(inputs, x_table, y_table):
    B, X, Y, D = inputs.shape
    return _sc_broadcast_add(x_table, y_table, B, X, Y, D)


# speedup vs baseline: 1.3961x; 1.0348x over previous
import functools

import jax
import jax.numpy as jnp
from jax import lax
from jax.experimental import pallas as pl
from jax.experimental.pallas import tpu as pltpu
from jax.experimental.pallas import tpu_sc as plsc
from jax._src.pallas import mpmd as plmpmd

_B_TEC = 42
_SCS_WINDOW = 8


def _sc_broadcast_add(x_table, y_table, B, X, Y, D):
    info = plsc.get_sparse_core_info()
    NC, NS, L = info.num_cores, info.num_subcores, info.num_lanes
    vmesh = plsc.VectorSubcoreMesh(core_axis_name="c", subcore_axis_name="s")
    smesh = plsc.ScalarSubcoreMesh(axis_name="c")
    b_tec = min(_B_TEC, B)
    xh = X // NC

    @functools.partial(
        pl.kernel,
        mesh=vmesh,
        out_type=jax.ShapeDtypeStruct((X, Y, D), jnp.float32),
        scratch_types=[
            pltpu.VMEM((Y, D), jnp.float32),
            pltpu.VMEM((1, D), jnp.float32),
            pltpu.VMEM((Y, D), jnp.float32),
        ],
    )
    def make_blocks(x_hbm, y_hbm, blocks_hbm, yblk, xrow, blk):
        w = lax.axis_index("s") * NC + lax.axis_index("c")
        pltpu.sync_copy(y_hbm.at[pl.ds(0, Y)], yblk)
        pltpu.sync_copy(x_hbm.at[pl.ds(w, 1)], xrow)

        def row(y, carry):
            for d0 in range(D // L):
                sl = pl.ds(d0 * L, L)
                blk[y, sl] = yblk[y, sl] + xrow[0, sl]
            return carry

        lax.fori_loop(0, Y, row, 0)
        pltpu.sync_copy(blk, blocks_hbm.at[w])

    blocks = make_blocks(x_table, y_table)

    def tec_fn(blocks_hbm, out_hbm, sp_blocks, scs_sem, tec_blk, tec_sem):
        del sp_blocks, scs_sem
        w = lax.axis_index("s") * NC + lax.axis_index("c")
        pltpu.sync_copy(blocks_hbm.at[w], tec_blk)
        copies = [
            pltpu.async_copy(tec_blk, out_hbm.at[b, w], tec_sem)
            for b in range(b_tec)
        ]
        for cp in copies:
            cp.wait()

    def scs_fn(blocks_hbm, out_hbm, sp_blocks, scs_sem, tec_blk, tec_sem):
        del tec_blk, tec_sem
        c = lax.axis_index("c")
        for i in range(xh):
            pltpu.sync_copy(blocks_hbm.at[i * NC + c], sp_blocks.at[i])
        handles = []
        for i in range(xh):
            x = i * NC + c
            for b in range(b_tec, B):
                if len(handles) >= _SCS_WINDOW:
                    handles[len(handles) - _SCS_WINDOW].wait()
                handles.append(
                    pltpu.async_copy(sp_blocks.at[i], out_hbm.at[b, x], scs_sem)
                )
        for h in handles[-_SCS_WINDOW:]:
            h.wait()

    run = plmpmd.mpmd_map(
        [(smesh, scs_fn), (vmesh, tec_fn)],
        out_types=[jax.ShapeDtypeStruct((B, X, Y, D), jnp.float32)],
        scratch_types=[
            pltpu.VMEM_SHARED((xh, Y, D), jnp.float32),
            pltpu.SemaphoreType.DMA @ smesh,
            (pltpu.VMEM @ vmesh)((Y, D), jnp.float32),
            pltpu.SemaphoreType.DMA @ vmesh,
        ],
    )
    return run(blocks)[0]


def kernel(inputs, x_table, y_table):
    B, X, Y, D = inputs.shape
    return _sc_broadcast_add(x_table, y_table, B, X, Y, D)


# speedup vs baseline: 1.6325x; 1.1693x over previous
import jax
import jax.numpy as jnp
from jax import lax
from jax.experimental import pallas as pl
from jax.experimental.pallas import tpu as pltpu
from jax.experimental.pallas import tpu_sc as plsc
from jax._src.pallas import mpmd as plmpmd

_B_TEC = 43
_SCS_WINDOW = 12


def _sc_broadcast_add(x_table, y_table, B, X, Y, D):
    info = plsc.get_sparse_core_info()
    NC, NS, L = info.num_cores, info.num_subcores, info.num_lanes
    vmesh = plsc.VectorSubcoreMesh(core_axis_name="c", subcore_axis_name="s")
    smesh = plsc.ScalarSubcoreMesh(axis_name="c")
    b_tec = min(_B_TEC, B)

    def tec_fn(x_hbm, y_hbm, out_hbm, sp_blocks, ready_sem, scs_sem,
               yblk, xrow, blk, tec_sem):
        del scs_sem
        c = lax.axis_index("c")
        s = lax.axis_index("s")
        w = s * NC + c
        pltpu.sync_copy(y_hbm.at[pl.ds(0, Y)], yblk)
        pltpu.sync_copy(x_hbm.at[pl.ds(w, 1)], xrow)

        def row(y, carry):
            for d0 in range(D // L):
                sl = pl.ds(d0 * L, L)
                blk[y, sl] = yblk[y, sl] + xrow[0, sl]
            return carry

        lax.fori_loop(0, Y, row, 0)

        pltpu.sync_copy(blk, sp_blocks.at[s])
        pltpu.semaphore_signal(ready_sem, 1)

        copies = [
            pltpu.async_copy(blk, out_hbm.at[b, w], tec_sem)
            for b in range(b_tec)
        ]
        for cp in copies:
            cp.wait()

    def scs_fn(x_hbm, y_hbm, out_hbm, sp_blocks, ready_sem, scs_sem,
               yblk, xrow, blk, tec_sem):
        del x_hbm, y_hbm, yblk, xrow, blk, tec_sem
        c = lax.axis_index("c")
        pltpu.semaphore_wait(ready_sem, NS)
        handles = []
        for i in range(NS):
            x = i * NC + c
            for b in range(b_tec, B):
                if len(handles) >= _SCS_WINDOW:
                    handles[len(handles) - _SCS_WINDOW].wait()
                handles.append(
                    pltpu.async_copy(sp_blocks.at[i], out_hbm.at[b, x], scs_sem)
                )
        for h in handles[-_SCS_WINDOW:]:
            h.wait()

    run = plmpmd.mpmd_map(
        [(smesh, scs_fn), (vmesh, tec_fn)],
        out_types=[jax.ShapeDtypeStruct((B, X, Y, D), jnp.float32)],
        scratch_types=[
            pltpu.VMEM_SHARED((NS, Y, D), jnp.float32),
            pltpu.SemaphoreType.REGULAR @ smesh,
            pltpu.SemaphoreType.DMA @ smesh,
            (pltpu.VMEM @ vmesh)((Y, D), jnp.float32),
            (pltpu.VMEM @ vmesh)((1, D), jnp.float32),
            (pltpu.VMEM @ vmesh)((Y, D), jnp.float32),
            pltpu.SemaphoreType.DMA @ vmesh,
        ],
    )
    return run(x_table, y_table)[0]


def kernel(inputs, x_table, y_table):
    B, X, Y, D = inputs.shape
    return _sc_broadcast_add(x_table, y_table, B, X, Y, D)


# speedup vs baseline: 1.6360x; 1.0021x over previous
import jax
import jax.numpy as jnp
from jax import lax
from jax.experimental import pallas as pl
from jax.experimental.pallas import tpu as pltpu
from jax.experimental.pallas import tpu_sc as plsc
from jax._src.pallas import mpmd as plmpmd

_B_TEC = 46
_SCS_WINDOW = 12


def _sc_broadcast_add(x_table, y_table, B, X, Y, D):
    info = plsc.get_sparse_core_info()
    NC, NS, L = info.num_cores, info.num_subcores, info.num_lanes
    vmesh = plsc.VectorSubcoreMesh(core_axis_name="c", subcore_axis_name="s")
    smesh = plsc.ScalarSubcoreMesh(axis_name="c")
    b_tec = min(_B_TEC, B)

    def tec_fn(x_hbm, y_hbm, out_hbm, sp_blocks, ready_sem, scs_sem,
               yblk, xrow, blk, tec_sem):
        del scs_sem
        c = lax.axis_index("c")
        s = lax.axis_index("s")
        w = s * NC + c
        pltpu.sync_copy(y_hbm.at[pl.ds(0, Y)], yblk)
        pltpu.sync_copy(x_hbm.at[pl.ds(w, 1)], xrow)

        def row(y, carry):
            for d0 in range(D // L):
                sl = pl.ds(d0 * L, L)
                blk[y, sl] = yblk[y, sl] + xrow[0, sl]
            return carry

        lax.fori_loop(0, Y, row, 0)

        pltpu.sync_copy(blk, sp_blocks.at[s])
        pltpu.semaphore_signal(ready_sem, 1)

        copies = [
            pltpu.async_copy(blk, out_hbm.at[b, w], tec_sem)
            for b in range(b_tec)
        ]
        for cp in copies:
            cp.wait()

    def scs_fn(x_hbm, y_hbm, out_hbm, sp_blocks, ready_sem, scs_sem,
               yblk, xrow, blk, tec_sem):
        del x_hbm, y_hbm, yblk, xrow, blk, tec_sem
        c = lax.axis_index("c")
        pltpu.semaphore_wait(ready_sem, NS)
        handles = []
        for i in range(NS):
            x = i * NC + c
            for b in range(b_tec, B):
                if len(handles) >= _SCS_WINDOW:
                    handles[len(handles) - _SCS_WINDOW].wait()
                handles.append(
                    pltpu.async_copy(sp_blocks.at[i], out_hbm.at[b, x], scs_sem)
                )
        for h in handles[-_SCS_WINDOW:]:
            h.wait()

    run = plmpmd.mpmd_map(
        [(smesh, scs_fn), (vmesh, tec_fn)],
        out_types=[jax.ShapeDtypeStruct((B, X, Y, D), jnp.float32)],
        scratch_types=[
            pltpu.VMEM_SHARED((NS, Y, D), jnp.float32),
            pltpu.SemaphoreType.REGULAR @ smesh,
            pltpu.SemaphoreType.DMA @ smesh,
            (pltpu.VMEM @ vmesh)((Y, D), jnp.float32),
            (pltpu.VMEM @ vmesh)((1, D), jnp.float32),
            (pltpu.VMEM @ vmesh)((Y, D), jnp.float32),
            pltpu.SemaphoreType.DMA @ vmesh,
        ],
    )
    return run(x_table, y_table)[0]


def kernel(inputs, x_table, y_table):
    B, X, Y, D = inputs.shape
    return _sc_broadcast_add(x_table, y_table, B, X, Y, D)


# speedup vs baseline: 1.6588x; 1.0140x over previous
import functools

import jax
import jax.numpy as jnp
from jax import lax
from jax.experimental import pallas as pl
from jax.experimental.pallas import tpu as pltpu
from jax.experimental.pallas import tpu_sc as plsc


def _sc_broadcast_add(x_table, y_table, B, X, Y, D):
    info = plsc.get_sparse_core_info()
    NC, NS, L = info.num_cores, info.num_subcores, info.num_lanes
    mesh = plsc.VectorSubcoreMesh(core_axis_name="c", subcore_axis_name="s")

    @functools.partial(
        pl.kernel,
        mesh=mesh,
        out_type=jax.ShapeDtypeStruct((B, X, Y, D), jnp.float32),
        scratch_types=[
            pltpu.VMEM((Y, D), jnp.float32),
            pltpu.VMEM((1, D), jnp.float32),
            pltpu.VMEM((Y, D), jnp.float32),
            pltpu.SemaphoreType.DMA,
        ],
    )
    def k(x_hbm, y_hbm, out_hbm, yblk, xrow, blk, sem):
        wid = lax.axis_index("s") * NC + lax.axis_index("c")
        x = wid
        pltpu.sync_copy(y_hbm.at[pl.ds(0, Y)], yblk)
        pltpu.sync_copy(x_hbm.at[pl.ds(x, 1)], xrow)

        def row(y, carry):
            for d0 in range(D // L):
                sl = pl.ds(d0 * L, L)
                blk[y, sl] = yblk[y, sl] + xrow[0, sl]
            return carry

        lax.fori_loop(0, Y, row, 0)

        copies = [
            pltpu.async_copy(blk, out_hbm.at[b, x], sem) for b in range(B)
        ]
        for c in copies:
            c.wait()

    return k(x_table, y_table)


def kernel(inputs, x_table, y_table):
    B, X, Y, D = inputs.shape
    return _sc_broadcast_add(x_table, y_table, B, X, Y, D)


# speedup vs baseline: 1.6667x; 1.0048x over previous
import functools

import jax
import jax.numpy as jnp
from jax import lax
from jax.experimental import pallas as pl
from jax.experimental.pallas import tpu as pltpu
from jax.experimental.pallas import tpu_sc as plsc


def _sc_broadcast_add(x_table, y_table, B, X, Y, D):
    info = plsc.get_sparse_core_info()
    NC, NS, L = info.num_cores, info.num_subcores, info.num_lanes
    mesh = plsc.VectorSubcoreMesh(core_axis_name="c", subcore_axis_name="s")
    yh = Y // 2

    @functools.partial(
        pl.kernel,
        mesh=mesh,
        out_type=jax.ShapeDtypeStruct((B, X, Y, D), jnp.float32),
        scratch_types=[
            pltpu.VMEM((Y, D), jnp.float32),
            pltpu.VMEM((1, D), jnp.float32),
            pltpu.VMEM((Y, D), jnp.float32),
            pltpu.SemaphoreType.DMA,
        ],
    )
    def k(x_hbm, y_hbm, out_hbm, yblk, xrow, blk, sem):
        wid = lax.axis_index("s") * NC + lax.axis_index("c")
        x = wid
        pltpu.sync_copy(y_hbm.at[pl.ds(0, Y)], yblk)
        pltpu.sync_copy(x_hbm.at[pl.ds(x, 1)], xrow)

        def rows(y0, n):
            def row(y, carry):
                for d0 in range(D // L):
                    sl = pl.ds(d0 * L, L)
                    blk[y, sl] = yblk[y, sl] + xrow[0, sl]
                return carry

            lax.fori_loop(y0, y0 + n, row, 0)

        copies = []
        rows(0, yh)
        copies += [
            pltpu.async_copy(
                blk.at[pl.ds(0, yh)], out_hbm.at[b, x, pl.ds(0, yh)], sem
            )
            for b in range(B)
        ]
        rows(yh, Y - yh)
        copies += [
            pltpu.async_copy(
                blk.at[pl.ds(yh, Y - yh)],
                out_hbm.at[b, x, pl.ds(yh, Y - yh)],
                sem,
            )
            for b in range(B)
        ]
        for c in copies:
            c.wait()

    return k(x_table, y_table)


def kernel(inputs, x_table, y_table):
    B, X, Y, D = inputs.shape
    return _sc_broadcast_add(x_table, y_table, B, X, Y, D)
